# Initial kernel scaffold; baseline (speedup 1.0000x reference)
import jax
import jax.numpy as jnp
from jax.experimental import pallas as pl


def kernel(x, edge_index, W1, b1, W2, b2, W3, b3, W4, b4):
    raise NotImplementedError("write your pallas kernel here")


# speedup vs baseline: 14.7611x; 14.7611x over previous
import jax
import jax.numpy as jnp
from jax import lax
from jax.experimental import pallas as pl
from jax.experimental.pallas import tpu as pltpu
from jax.experimental.pallas import tpu_sc as plsc

N = 10000
N2 = 10240
E = 320000
DP = 128
NCORE = 2
NSUB = 16
NW = NCORE * NSUB
NBLK = 125
KE = 80
RPT = N2 // NSUB

assert NW * NBLK * KE == E

_mesh = plsc.VectorSubcoreMesh(core_axis_name="c", subcore_axis_name="s")


def _agg_body(z_hbm, srcb_hbm, dstb_hbm, out_hbm, src_v, dst_v, rows_v, acc_sh, sem):
    cid = lax.axis_index("c")
    sid = lax.axis_index("s")
    wid = cid * NSUB + sid
    rows = pl.ds(sid * RPT, RPT)

    pltpu.sync_copy(srcb_hbm.at[wid], src_v)
    pltpu.sync_copy(dstb_hbm.at[wid], dst_v)
    pltpu.sync_copy(z_hbm.at[rows], acc_sh.at[rows])
    plsc.subcore_barrier()

    def blk(j, carry):
        pltpu.async_copy(z_hbm.at[src_v.at[j]], rows_v, sem).wait()
        pltpu.sync_copy(rows_v, acc_sh.at[dst_v.at[j]], add=True)
        return carry

    lax.fori_loop(0, NBLK, blk, 0)
    plsc.subcore_barrier()
    pltpu.sync_copy(acc_sh.at[rows], out_hbm.at[pl.ds(cid * N2 + sid * RPT, RPT)])


_agg = pl.kernel(
    _agg_body,
    out_type=jax.ShapeDtypeStruct((NCORE * N2, DP), jnp.float32),
    mesh=_mesh,
    scratch_types=[
        pltpu.VMEM((NBLK, KE), jnp.int32),
        pltpu.VMEM((NBLK, KE), jnp.int32),
        pltpu.VMEM((KE, DP), jnp.float32),
        pltpu.VMEM_SHARED((N2, DP), jnp.float32),
        pltpu.SemaphoreType.DMA,
    ],
)


def _deg_body(ones_hbm, dstb_hbm, out_hbm, dst_v, rows_v, acc_sh):
    cid = lax.axis_index("c")
    sid = lax.axis_index("s")
    wid = cid * NSUB + sid
    rows = pl.ds(sid * RPT, RPT)

    pltpu.sync_copy(dstb_hbm.at[wid], dst_v)
    pltpu.sync_copy(ones_hbm.at[pl.ds(0, KE)], rows_v)
    pltpu.sync_copy(ones_hbm.at[rows], acc_sh.at[rows])
    plsc.subcore_barrier()

    def blk(j, carry):
        pltpu.sync_copy(rows_v, acc_sh.at[dst_v.at[j]], add=True)
        return carry

    lax.fori_loop(0, NBLK, blk, 0)
    plsc.subcore_barrier()
    pltpu.sync_copy(acc_sh.at[rows], out_hbm.at[pl.ds(cid * N2 + sid * RPT, RPT)])


_deg = pl.kernel(
    _deg_body,
    out_type=jax.ShapeDtypeStruct((NCORE * N2, 16), jnp.float32),
    mesh=_mesh,
    scratch_types=[
        pltpu.VMEM((NBLK, KE), jnp.int32),
        pltpu.VMEM((KE, 16), jnp.float32),
        pltpu.VMEM_SHARED((N2, 16), jnp.float32),
    ],
)


def _pad_cols(y, d_out):
    if d_out < DP:
        y = jnp.concatenate([y, jnp.zeros((y.shape[0], DP - d_out), jnp.float32)], axis=1)
    return y


def _tc_first(x, W1, deg2):
    d_out = W1.shape[1]

    def bdy(x_ref, w_ref, deg_ref, z_ref, dis_ref):
        deg = deg_ref[0:N, 0:1] + deg_ref[N2 : N2 + N, 0:1] - 1.0
        dis = 1.0 / jnp.sqrt(deg)
        dis_ref[...] = dis
        y = dis * jnp.dot(x_ref[...], w_ref[...], preferred_element_type=jnp.float32)
        z_ref[0:N, :] = _pad_cols(y, d_out)
        z_ref[N:N2, :] = jnp.zeros((N2 - N, DP), jnp.float32)

    return pl.pallas_call(
        bdy,
        out_shape=[
            jax.ShapeDtypeStruct((N2, DP), jnp.float32),
            jax.ShapeDtypeStruct((N, 1), jnp.float32),
        ],
    )(x, W1, deg2)


def _tc_mid(s2, z, dis, b, W, relu):
    d_in = W.shape[0]
    d_out = W.shape[1]

    def bdy(s_ref, z_ref, dis_ref, b_ref, w_ref, out_ref):
        agg = s_ref[0:N, 0:d_in] + s_ref[N2 : N2 + N, 0:d_in] - z_ref[0:N, 0:d_in]
        y = dis_ref[...] * agg + b_ref[...]
        if relu:
            y = jnp.maximum(y, 0.0)
        zn = dis_ref[...] * jnp.dot(y, w_ref[...], preferred_element_type=jnp.float32)
        out_ref[0:N, :] = _pad_cols(zn, d_out)
        out_ref[N:N2, :] = jnp.zeros((N2 - N, DP), jnp.float32)

    return pl.pallas_call(
        bdy,
        out_shape=jax.ShapeDtypeStruct((N2, DP), jnp.float32),
    )(s2, z, dis, b, W)


def _tc_latent(s2, z, dis, b, W):
    d_in = W.shape[0]
    d_out = W.shape[1]

    def bdy(s_ref, z_ref, dis_ref, b_ref, w_ref, lat_ref, zn_ref):
        agg = s_ref[0:N, 0:d_in] + s_ref[N2 : N2 + N, 0:d_in] - z_ref[0:N, 0:d_in]
        lat = dis_ref[...] * agg + b_ref[...]
        lat_ref[...] = lat
        zn = dis_ref[...] * jnp.dot(lat, w_ref[...], preferred_element_type=jnp.float32)
        zn_ref[0:N, :] = _pad_cols(zn, d_out)
        zn_ref[N:N2, :] = jnp.zeros((N2 - N, DP), jnp.float32)

    return pl.pallas_call(
        bdy,
        out_shape=[
            jax.ShapeDtypeStruct((N, d_in), jnp.float32),
            jax.ShapeDtypeStruct((N2, DP), jnp.float32),
        ],
    )(s2, z, dis, b, W)


def _tc_last(s2, z, dis, b):
    def bdy(s_ref, z_ref, dis_ref, b_ref, out_ref):
        agg = s_ref[0:N] + s_ref[N2 : N2 + N] - z_ref[0:N]
        out_ref[...] = dis_ref[...] * agg + b_ref[...]

    return pl.pallas_call(
        bdy,
        out_shape=jax.ShapeDtypeStruct((N, DP), jnp.float32),
    )(s2, z, dis, b)


def kernel(x, edge_index, W1, b1, W2, b2, W3, b3, W4, b4):
    src = edge_index[0].reshape(NW, NBLK, KE)
    dst = edge_index[1].reshape(NW, NBLK, KE)
    ones16 = jnp.ones((N2, 16), jnp.float32)

    deg2 = _deg(ones16, dst)
    z1, dis = _tc_first(x, W1, deg2)
    s1 = _agg(z1, src, dst)
    z2 = _tc_mid(s1, z1, dis, b1.reshape(1, -1), W2, relu=True)
    s2 = _agg(z2, src, dst)
    latent, z3 = _tc_latent(s2, z2, dis, b2.reshape(1, -1), W3)
    s3 = _agg(z3, src, dst)
    z4 = _tc_mid(s3, z3, dis, b3.reshape(1, -1), W4, relu=True)
    s4 = _agg(z4, src, dst)
    reconstructed = _tc_last(s4, z4, dis, b4.reshape(1, -1))
    return (reconstructed, latent)


# speedup vs baseline: 22.9439x; 1.5544x over previous
import jax
import jax.numpy as jnp
from jax import lax
from jax.experimental import pallas as pl
from jax.experimental.pallas import tpu as pltpu
from jax.experimental.pallas import tpu_sc as plsc

N = 10000
N2 = 10240
E = 320000
DP = 128
NCORE = 2
NSUB = 16
NW = NCORE * NSUB
NBLK = 125
KE = 80
G = 25
NG = NBLK // G
RPT = N2 // NSUB
NBLK_D = 125
KE_D = 80

assert NW * NBLK * KE == E
assert NW * NBLK_D * KE_D == E

_mesh = plsc.VectorSubcoreMesh(core_axis_name="c", subcore_axis_name="s")


def _agg_body(z_hbm, sdb_hbm, out_hbm, sd_v0, sd_v1, buf0, buf1, acc_sh, sem0, sem1, semi):
    cid = lax.axis_index("c")
    sid = lax.axis_index("s")
    wid = cid * NSUB + sid
    rows = pl.ds(sid * RPT, RPT)

    sd_bufs = (sd_v0, sd_v1)
    pltpu.async_copy(sdb_hbm.at[wid, pl.ds(0, G)], sd_v0, semi)
    pltpu.sync_copy(z_hbm.at[rows], acc_sh.at[rows])
    plsc.subcore_barrier()

    for g in range(NG):
        sd_v = sd_bufs[g % 2]
        pltpu.make_async_copy(sdb_hbm.at[wid, pl.ds(g * G, G)], sd_v, semi).wait()
        if g + 1 < NG:
            pltpu.async_copy(
                sdb_hbm.at[wid, pl.ds((g + 1) * G, G)], sd_bufs[(g + 1) % 2], semi
            )
        pltpu.async_copy(z_hbm.at[sd_v.at[0, 0]], buf0, sem0)

        def it(i, carry):
            j = 2 * i
            pltpu.async_copy(z_hbm.at[sd_v.at[j + 1, 0]], buf1, sem1)
            pltpu.make_async_copy(z_hbm.at[sd_v.at[j, 0]], buf0, sem0).wait()
            pltpu.sync_copy(buf0, acc_sh.at[sd_v.at[j, 1]], add=True)
            pltpu.async_copy(z_hbm.at[sd_v.at[j + 2, 0]], buf0, sem0)
            pltpu.make_async_copy(z_hbm.at[sd_v.at[j + 1, 0]], buf1, sem1).wait()
            pltpu.sync_copy(buf1, acc_sh.at[sd_v.at[j + 1, 1]], add=True)
            return carry

        lax.fori_loop(0, (G - 1) // 2, it, 0)
        pltpu.make_async_copy(z_hbm.at[sd_v.at[G - 1, 0]], buf0, sem0).wait()
        pltpu.sync_copy(buf0, acc_sh.at[sd_v.at[G - 1, 1]], add=True)

    plsc.subcore_barrier()
    pltpu.sync_copy(acc_sh.at[rows], out_hbm.at[pl.ds(cid * N2 + sid * RPT, RPT)])


_agg = pl.kernel(
    _agg_body,
    out_type=jax.ShapeDtypeStruct((NCORE * N2, DP), jnp.float32),
    mesh=_mesh,
    scratch_types=[
        pltpu.VMEM((G, 2, KE), jnp.int32),
        pltpu.VMEM((G, 2, KE), jnp.int32),
        pltpu.VMEM((KE, DP), jnp.float32),
        pltpu.VMEM((KE, DP), jnp.float32),
        pltpu.VMEM_SHARED((N2, DP), jnp.float32),
        pltpu.SemaphoreType.DMA,
        pltpu.SemaphoreType.DMA,
        pltpu.SemaphoreType.DMA,
    ],
)


def _deg_body(ones_hbm, dstb_hbm, out_hbm, dst_v, rows_v, acc_sh):
    cid = lax.axis_index("c")
    sid = lax.axis_index("s")
    wid = cid * NSUB + sid
    rows = pl.ds(sid * RPT, RPT)

    pltpu.sync_copy(dstb_hbm.at[wid], dst_v)
    pltpu.sync_copy(ones_hbm.at[pl.ds(0, KE_D)], rows_v)
    pltpu.sync_copy(ones_hbm.at[rows], acc_sh.at[rows])
    plsc.subcore_barrier()

    def blk(j, carry):
        pltpu.sync_copy(rows_v, acc_sh.at[dst_v.at[j]], add=True)
        return carry

    lax.fori_loop(0, NBLK_D, blk, 0)
    plsc.subcore_barrier()
    pltpu.sync_copy(acc_sh.at[rows], out_hbm.at[pl.ds(cid * N2 + sid * RPT, RPT)])


_deg = pl.kernel(
    _deg_body,
    out_type=jax.ShapeDtypeStruct((NCORE * N2, 16), jnp.float32),
    mesh=_mesh,
    scratch_types=[
        pltpu.VMEM((NBLK_D, KE_D), jnp.int32),
        pltpu.VMEM((KE_D, 16), jnp.float32),
        pltpu.VMEM_SHARED((N2, 16), jnp.float32),
    ],
)


def _pad_cols(y, d_out):
    if d_out < DP:
        y = jnp.concatenate([y, jnp.zeros((y.shape[0], DP - d_out), jnp.float32)], axis=1)
    return y


def _tc_first(x, W1, deg2):
    d_out = W1.shape[1]

    def bdy(x_ref, w_ref, deg_ref, z_ref, dis_ref):
        deg = deg_ref[0:N, 0:1] + deg_ref[N2 : N2 + N, 0:1] - 1.0
        dis = 1.0 / jnp.sqrt(deg)
        dis_ref[...] = dis
        y = dis * jnp.dot(x_ref[...], w_ref[...], preferred_element_type=jnp.float32)
        z_ref[0:N, :] = _pad_cols(y, d_out)
        z_ref[N:N2, :] = jnp.zeros((N2 - N, DP), jnp.float32)

    return pl.pallas_call(
        bdy,
        out_shape=[
            jax.ShapeDtypeStruct((N2, DP), jnp.float32),
            jax.ShapeDtypeStruct((N, 1), jnp.float32),
        ],
    )(x, W1, deg2)


def _tc_mid(s2, z, dis, b, W, relu):
    d_in = W.shape[0]
    d_out = W.shape[1]

    def bdy(s_ref, z_ref, dis_ref, b_ref, w_ref, out_ref):
        agg = s_ref[0:N, 0:d_in] + s_ref[N2 : N2 + N, 0:d_in] - z_ref[0:N, 0:d_in]
        y = dis_ref[...] * agg + b_ref[...]
        if relu:
            y = jnp.maximum(y, 0.0)
        zn = dis_ref[...] * jnp.dot(y, w_ref[...], preferred_element_type=jnp.float32)
        out_ref[0:N, :] = _pad_cols(zn, d_out)
        out_ref[N:N2, :] = jnp.zeros((N2 - N, DP), jnp.float32)

    return pl.pallas_call(
        bdy,
        out_shape=jax.ShapeDtypeStruct((N2, DP), jnp.float32),
    )(s2, z, dis, b, W)


def _tc_latent(s2, z, dis, b, W):
    d_in = W.shape[0]
    d_out = W.shape[1]

    def bdy(s_ref, z_ref, dis_ref, b_ref, w_ref, lat_ref, zn_ref):
        agg = s_ref[0:N, 0:d_in] + s_ref[N2 : N2 + N, 0:d_in] - z_ref[0:N, 0:d_in]
        lat = dis_ref[...] * agg + b_ref[...]
        lat_ref[...] = lat
        zn = dis_ref[...] * jnp.dot(lat, w_ref[...], preferred_element_type=jnp.float32)
        zn_ref[0:N, :] = _pad_cols(zn, d_out)
        zn_ref[N:N2, :] = jnp.zeros((N2 - N, DP), jnp.float32)

    return pl.pallas_call(
        bdy,
        out_shape=[
            jax.ShapeDtypeStruct((N, d_in), jnp.float32),
            jax.ShapeDtypeStruct((N2, DP), jnp.float32),
        ],
    )(s2, z, dis, b, W)


def _tc_last(s2, z, dis, b):
    def bdy(s_ref, z_ref, dis_ref, b_ref, out_ref):
        agg = s_ref[0:N] + s_ref[N2 : N2 + N] - z_ref[0:N]
        out_ref[...] = dis_ref[...] * agg + b_ref[...]

    return pl.pallas_call(
        bdy,
        out_shape=jax.ShapeDtypeStruct((N, DP), jnp.float32),
    )(s2, z, dis, b)


def kernel(x, edge_index, W1, b1, W2, b2, W3, b3, W4, b4):
    src = edge_index[0].reshape(NW, NBLK, KE)
    dst = edge_index[1].reshape(NW, NBLK, KE)
    sd = jnp.stack([src, dst], axis=2)
    dst_d = edge_index[1].reshape(NW, NBLK_D, KE_D)
    ones16 = jnp.ones((N2, 16), jnp.float32)

    deg2 = _deg(ones16, dst_d)
    z1, dis = _tc_first(x, W1, deg2)
    s1 = _agg(z1, sd)
    z2 = _tc_mid(s1, z1, dis, b1.reshape(1, -1), W2, relu=True)
    s2 = _agg(z2, sd)
    latent, z3 = _tc_latent(s2, z2, dis, b2.reshape(1, -1), W3)
    s3 = _agg(z3, sd)
    z4 = _tc_mid(s3, z3, dis, b3.reshape(1, -1), W4, relu=True)
    s4 = _agg(z4, sd)
    reconstructed = _tc_last(s4, z4, dis, b4.reshape(1, -1))
    return (reconstructed, latent)


# speedup vs baseline: 23.0627x; 1.0052x over previous
import functools

import jax
import jax.numpy as jnp
from jax import lax
from jax.experimental import pallas as pl
from jax.experimental.pallas import tpu as pltpu
from jax.experimental.pallas import tpu_sc as plsc

N = 10000
N2 = 10240
E = 320000
DP = 128
NCORE = 2
NSUB = 16
NW = NCORE * NSUB
NBLK = 125
KE = 80
G = 25
NG = NBLK // G
RPT = N2 // NSUB
NBLK_D = 125
KE_D = 80

assert NW * NBLK * KE == E
assert NW * NBLK_D * KE_D == E

_mesh = plsc.VectorSubcoreMesh(core_axis_name="c", subcore_axis_name="s")


def _agg_body(z_hbm, zer_hbm, sdb_hbm, out_hbm, sd_v0, sd_v1, buf0, buf1, acc_sh, sem0, sem1, semi):
    cid = lax.axis_index("c")
    sid = lax.axis_index("s")
    wid = cid * NSUB + sid
    rows = pl.ds(sid * RPT, RPT)

    sd_bufs = (sd_v0, sd_v1)
    pltpu.async_copy(sdb_hbm.at[wid, pl.ds(0, G)], sd_v0, semi)

    @pl.when(cid == 0)
    def _():
        pltpu.sync_copy(z_hbm.at[rows], acc_sh.at[rows])

    @pl.when(cid == 1)
    def _():
        pltpu.sync_copy(zer_hbm.at[rows], acc_sh.at[rows])

    plsc.subcore_barrier()

    for g in range(NG):
        sd_v = sd_bufs[g % 2]
        pltpu.make_async_copy(sdb_hbm.at[wid, pl.ds(g * G, G)], sd_v, semi).wait()
        if g + 1 < NG:
            pltpu.async_copy(
                sdb_hbm.at[wid, pl.ds((g + 1) * G, G)], sd_bufs[(g + 1) % 2], semi
            )
        pltpu.async_copy(z_hbm.at[sd_v.at[0, 0]], buf0, sem0)

        def it(i, carry):
            j = 2 * i
            pltpu.async_copy(z_hbm.at[sd_v.at[j + 1, 0]], buf1, sem1)
            pltpu.make_async_copy(z_hbm.at[sd_v.at[j, 0]], buf0, sem0).wait()
            pltpu.sync_copy(buf0, acc_sh.at[sd_v.at[j, 1]], add=True)
            pltpu.async_copy(z_hbm.at[sd_v.at[j + 2, 0]], buf0, sem0)
            pltpu.make_async_copy(z_hbm.at[sd_v.at[j + 1, 0]], buf1, sem1).wait()
            pltpu.sync_copy(buf1, acc_sh.at[sd_v.at[j + 1, 1]], add=True)
            return carry

        lax.fori_loop(0, (G - 1) // 2, it, 0)
        pltpu.make_async_copy(z_hbm.at[sd_v.at[G - 1, 0]], buf0, sem0).wait()
        pltpu.sync_copy(buf0, acc_sh.at[sd_v.at[G - 1, 1]], add=True)

    plsc.subcore_barrier()
    pltpu.sync_copy(acc_sh.at[rows], out_hbm.at[pl.ds(cid * N2 + sid * RPT, RPT)])


_agg = pl.kernel(
    _agg_body,
    out_type=jax.ShapeDtypeStruct((NCORE * N2, DP), jnp.float32),
    mesh=_mesh,
    scratch_types=[
        pltpu.VMEM((G, 2, KE), jnp.int32),
        pltpu.VMEM((G, 2, KE), jnp.int32),
        pltpu.VMEM((KE, DP), jnp.float32),
        pltpu.VMEM((KE, DP), jnp.float32),
        pltpu.VMEM_SHARED((N2, DP), jnp.float32),
        pltpu.SemaphoreType.DMA,
        pltpu.SemaphoreType.DMA,
        pltpu.SemaphoreType.DMA,
    ],
)


def _deg_body(ones_hbm, dstb_hbm, out_hbm, dst_v, rows_v, acc_sh):
    cid = lax.axis_index("c")
    sid = lax.axis_index("s")
    wid = cid * NSUB + sid
    rows = pl.ds(sid * RPT, RPT)

    pltpu.sync_copy(dstb_hbm.at[wid], dst_v)
    pltpu.sync_copy(ones_hbm.at[pl.ds(0, KE_D)], rows_v)
    pltpu.sync_copy(ones_hbm.at[rows], acc_sh.at[rows])
    plsc.subcore_barrier()

    def blk(j, carry):
        pltpu.sync_copy(rows_v, acc_sh.at[dst_v.at[j]], add=True)
        return carry

    lax.fori_loop(0, NBLK_D, blk, 0)
    plsc.subcore_barrier()
    pltpu.sync_copy(acc_sh.at[rows], out_hbm.at[pl.ds(cid * N2 + sid * RPT, RPT)])


_deg = pl.kernel(
    _deg_body,
    out_type=jax.ShapeDtypeStruct((NCORE * N2, 16), jnp.float32),
    mesh=_mesh,
    scratch_types=[
        pltpu.VMEM((NBLK_D, KE_D), jnp.int32),
        pltpu.VMEM((KE_D, 16), jnp.float32),
        pltpu.VMEM_SHARED((N2, 16), jnp.float32),
    ],
)


def _pad_cols(y, d_out):
    if d_out < DP:
        y = jnp.concatenate([y, jnp.zeros((y.shape[0], DP - d_out), jnp.float32)], axis=1)
    return y


def _tc_first(x, W1, deg2):
    d_out = W1.shape[1]

    def bdy(x_ref, w_ref, deg_ref, z_ref, dis_ref):
        deg = deg_ref[0:N, 0:1] + deg_ref[N2 : N2 + N, 0:1] - 1.0
        dis = 1.0 / jnp.sqrt(deg)
        dis_ref[...] = dis
        y = dis * jnp.dot(x_ref[...], w_ref[...], preferred_element_type=jnp.float32)
        z_ref[0:N, :] = _pad_cols(y, d_out)
        z_ref[N:N2, :] = jnp.zeros((N2 - N, DP), jnp.float32)

    return pl.pallas_call(
        bdy,
        out_shape=[
            jax.ShapeDtypeStruct((N2, DP), jnp.float32),
            jax.ShapeDtypeStruct((N, 1), jnp.float32),
        ],
    )(x, W1, deg2)


def _tc_mid(s2, dis, b, W, relu):
    d_in = W.shape[0]
    d_out = W.shape[1]

    def bdy(s_ref, dis_ref, b_ref, w_ref, out_ref):
        agg = s_ref[0:N, 0:d_in] + s_ref[N2 : N2 + N, 0:d_in]
        y = dis_ref[...] * agg + b_ref[...]
        if relu:
            y = jnp.maximum(y, 0.0)
        zn = dis_ref[...] * jnp.dot(y, w_ref[...], preferred_element_type=jnp.float32)
        out_ref[0:N, :] = _pad_cols(zn, d_out)
        out_ref[N:N2, :] = jnp.zeros((N2 - N, DP), jnp.float32)

    return pl.pallas_call(
        bdy,
        out_shape=jax.ShapeDtypeStruct((N2, DP), jnp.float32),
    )(s2, dis, b, W)


def _tc_latent(s2, dis, b, W):
    d_in = W.shape[0]
    d_out = W.shape[1]

    def bdy(s_ref, dis_ref, b_ref, w_ref, lat_ref, zn_ref):
        agg = s_ref[0:N, 0:d_in] + s_ref[N2 : N2 + N, 0:d_in]
        lat = dis_ref[...] * agg + b_ref[...]
        lat_ref[...] = lat
        zn = dis_ref[...] * jnp.dot(lat, w_ref[...], preferred_element_type=jnp.float32)
        zn_ref[0:N, :] = _pad_cols(zn, d_out)
        zn_ref[N:N2, :] = jnp.zeros((N2 - N, DP), jnp.float32)

    return pl.pallas_call(
        bdy,
        out_shape=[
            jax.ShapeDtypeStruct((N, d_in), jnp.float32),
            jax.ShapeDtypeStruct((N2, DP), jnp.float32),
        ],
    )(s2, dis, b, W)


def _tc_last(s2, dis, b):
    def bdy(s_ref, dis_ref, b_ref, out_ref):
        agg = s_ref[0:N] + s_ref[N2 : N2 + N]
        out_ref[...] = dis_ref[...] * agg + b_ref[...]

    return pl.pallas_call(
        bdy,
        out_shape=jax.ShapeDtypeStruct((N, DP), jnp.float32),
    )(s2, dis, b)


def kernel(x, edge_index, W1, b1, W2, b2, W3, b3, W4, b4):
    src = edge_index[0].reshape(NW, NBLK, KE)
    dst = edge_index[1].reshape(NW, NBLK, KE)
    sd = jnp.stack([src, dst], axis=2)
    dst_d = edge_index[1].reshape(NW, NBLK_D, KE_D)
    ones16 = jnp.ones((N2, 16), jnp.float32)
    zer = jnp.zeros((N2, DP), jnp.float32)

    deg2 = _deg(ones16, dst_d)
    z1, dis = _tc_first(x, W1, deg2)
    s1 = _agg(z1, zer, sd)
    z2 = _tc_mid(s1, dis, b1.reshape(1, -1), W2, relu=True)
    s2 = _agg(z2, zer, sd)
    latent, z3 = _tc_latent(s2, dis, b2.reshape(1, -1), W3)
    s3 = _agg(z3, zer, sd)
    z4 = _tc_mid(s3, dis, b3.reshape(1, -1), W4, relu=True)
    s4 = _agg(z4, zer, sd)
    reconstructed = _tc_last(s4, dis, b4.reshape(1, -1))
    return (reconstructed, latent)
